# baseline (speedup 1.0000x reference)
import jax
import jax.numpy as jnp
from jax import lax
from jax.experimental import pallas as pl
from jax.experimental.pallas import tpu as pltpu
from jax.experimental.pallas import tpu_sc as plsc

DIMS = 4
W = 72
NPIX = W * W
NWORK = 27
CHUNK = NPIX // NWORK
NVEC = CHUNK // 16
RWIN = 10.0


def _iou_body(out_flat_hbm, hwf_hbm, small_hbm, out_hbm,
              feat_v, hw_v, out_v, small_v, sem):
    wid = lax.axis_index("s") * 2 + lax.axis_index("c")

    @pl.when(wid < NWORK)
    def _():
        base = wid * CHUNK
        cps = [
            pltpu.async_copy(small_hbm, small_v, sem),
            pltpu.async_copy(hwf_hbm.at[pl.ds(base, CHUNK)],
                             hw_v.at[pl.ds(0, CHUNK)], sem),
            pltpu.async_copy(hwf_hbm.at[pl.ds(NPIX + base, CHUNK)],
                             hw_v.at[pl.ds(CHUNK, CHUNK)], sem),
        ]
        for c in range(DIMS):
            cps.append(pltpu.async_copy(
                out_flat_hbm.at[pl.ds(c * NPIX + base, CHUNK)],
                feat_v.at[pl.ds(c * CHUNK, CHUNK)], sem))
        for cp in cps:
            cp.wait()
        chv = small_v[pl.ds(0, 16)]
        cwv = small_v[pl.ds(16, 16)]
        rad = small_v[pl.ds(32, 16)]
        tl = small_v[pl.ds(48, 16)]
        tr = small_v[pl.ds(64, 16)]
        tt = small_v[pl.ds(80, 16)]
        tb = small_v[pl.ds(96, 16)]
        for j in range(NVEC):
            hf = hw_v[pl.ds(j * 16, 16)]
            wf = hw_v[pl.ds(CHUNK + j * 16, 16)]
            rhf = hf - chv
            rwf = wf - cwv
            rha = jnp.abs(rhf)
            rwa = jnp.abs(rwf)
            inwin = (rha <= RWIN) & (rwa <= RWIN) & (rha <= rad) & (rwa <= rad)
            twl = tl + rwf
            twr = tr - rwf
            tht = tt + rhf
            thb = tb - rhf
            okt = (twl >= 0.0) & (twr >= 0.0) & (tht >= 0.0) & (thb >= 0.0)
            p0 = feat_v[pl.ds(0 * CHUNK + j * 16, 16)]
            p1 = feat_v[pl.ds(1 * CHUNK + j * 16, 16)]
            p2 = feat_v[pl.ds(2 * CHUNK + j * 16, 16)]
            p3 = feat_v[pl.ds(3 * CHUNK + j * 16, 16)]
            t_area = (twl + twr) * (tht + thb)
            p_area = (p0 + p1) * (p2 + p3)
            w_int = jnp.minimum(p0, twl) + jnp.minimum(p1, twr)
            h_int = jnp.minimum(p3, thb) + jnp.minimum(p2, tht)
            a_int = w_int * h_int
            a_un = t_area + p_area - a_int
            iou = (a_int + 1.0) / (a_un + 1.0)
            out_v[pl.ds(j * 16, 16)] = jnp.where(inwin & okt, iou, 0.0)
        pltpu.sync_copy(out_v, out_hbm.at[pl.ds(base, CHUNK)])


def kernel(output, ind, target, radius):
    out_flat = output.reshape(-1)
    pix = jnp.arange(NPIX, dtype=jnp.float32)
    hwf = jnp.concatenate([jnp.floor(pix / W), pix % W])
    cen = ind.reshape(-1)[0].astype(jnp.float32)
    t4 = target.reshape(-1, DIMS)[0]
    small = jnp.concatenate([
        jnp.broadcast_to(jnp.floor(cen / W), (16,)),
        jnp.broadcast_to(cen % W, (16,)),
        jnp.broadcast_to(jnp.asarray(radius, jnp.float32), (16,)),
        jnp.broadcast_to(t4[0], (16,)),
        jnp.broadcast_to(t4[1], (16,)),
        jnp.broadcast_to(t4[2], (16,)),
        jnp.broadcast_to(t4[3], (16,)),
    ])
    mesh = plsc.VectorSubcoreMesh(core_axis_name="c", subcore_axis_name="s")
    iou_flat = pl.kernel(
        _iou_body,
        mesh=mesh,
        out_type=jax.ShapeDtypeStruct((NPIX,), jnp.float32),
        scratch_types=[
            pltpu.VMEM((DIMS * CHUNK,), jnp.float32),
            pltpu.VMEM((2 * CHUNK,), jnp.float32),
            pltpu.VMEM((CHUNK,), jnp.float32),
            pltpu.VMEM((7 * 16,), jnp.float32),
            pltpu.SemaphoreType.DMA,
        ],
    )(out_flat, hwf, small)
    return iou_flat.reshape(W, W)


# speedup vs baseline: 3.6336x; 3.6336x over previous
import jax
import jax.numpy as jnp
from jax import lax
from jax.experimental import pallas as pl
from jax.experimental.pallas import tpu as pltpu
from jax.experimental.pallas import tpu_sc as plsc

DIMS = 4
W = 72
NPIX = W * W
NWORK = 27
CHUNK = NPIX // NWORK
NVEC = CHUNK // 16
RWIN = 10.0


def _iou_body(feat_hbm, hwf_hbm, small_hbm, out_hbm,
              feat_v, hw_v, out_v, small_v, sem):
    wid = lax.axis_index("s") * 2 + lax.axis_index("c")

    @pl.when(wid < NWORK)
    def _():
        base = wid * CHUNK
        cps = [
            pltpu.async_copy(small_hbm, small_v, sem),
            pltpu.async_copy(hwf_hbm.at[pl.ds(base, CHUNK)],
                             hw_v.at[pl.ds(0, CHUNK)], sem),
            pltpu.async_copy(hwf_hbm.at[pl.ds(NPIX + base, CHUNK)],
                             hw_v.at[pl.ds(CHUNK, CHUNK)], sem),
        ]
        for c in range(DIMS):
            cps.append(pltpu.async_copy(
                feat_hbm.at[pl.ds(c * NPIX + base, CHUNK)],
                feat_v.at[pl.ds(c * CHUNK, CHUNK)], sem))
        for cp in cps:
            cp.wait()
        chv = small_v[pl.ds(0, 16)]
        cwv = small_v[pl.ds(16, 16)]
        rad = small_v[pl.ds(32, 16)]
        tl = small_v[pl.ds(48, 16)]
        tr = small_v[pl.ds(64, 16)]
        tt = small_v[pl.ds(80, 16)]
        tb = small_v[pl.ds(96, 16)]
        for j in range(NVEC):
            hf = hw_v[pl.ds(j * 16, 16)]
            wf = hw_v[pl.ds(CHUNK + j * 16, 16)]
            rhf = hf - chv
            rwf = wf - cwv
            rha = jnp.abs(rhf)
            rwa = jnp.abs(rwf)
            inwin = (rha <= RWIN) & (rwa <= RWIN) & (rha <= rad) & (rwa <= rad)
            twl = tl + rwf
            twr = tr - rwf
            tht = tt + rhf
            thb = tb - rhf
            okt = (twl >= 0.0) & (twr >= 0.0) & (tht >= 0.0) & (thb >= 0.0)
            p0 = feat_v[pl.ds(0 * CHUNK + j * 16, 16)]
            p1 = feat_v[pl.ds(1 * CHUNK + j * 16, 16)]
            p2 = feat_v[pl.ds(2 * CHUNK + j * 16, 16)]
            p3 = feat_v[pl.ds(3 * CHUNK + j * 16, 16)]
            t_area = (twl + twr) * (tht + thb)
            p_area = (p0 + p1) * (p2 + p3)
            w_int = jnp.minimum(p0, twl) + jnp.minimum(p1, twr)
            h_int = jnp.minimum(p3, thb) + jnp.minimum(p2, tht)
            a_int = w_int * h_int
            a_un = t_area + p_area - a_int
            iou = (a_int + 1.0) / (a_un + 1.0)
            out_v[pl.ds(j * 16, 16)] = jnp.where(inwin & okt, iou, 0.0)
        pltpu.sync_copy(out_v, out_hbm.at[pl.ds(base, CHUNK)])


def kernel(output, ind, target, radius):
    feat = output.reshape(-1, DIMS * NPIX)[0]
    pix = jnp.arange(NPIX, dtype=jnp.float32)
    hwf = jnp.concatenate([jnp.floor(pix / W), pix % W])
    cen = ind.reshape(-1)[0].astype(jnp.float32)
    t4 = target.reshape(-1, DIMS)[0]
    small = jnp.concatenate([
        jnp.broadcast_to(jnp.floor(cen / W), (16,)),
        jnp.broadcast_to(cen % W, (16,)),
        jnp.broadcast_to(jnp.asarray(radius, jnp.float32), (16,)),
        jnp.broadcast_to(t4[0], (16,)),
        jnp.broadcast_to(t4[1], (16,)),
        jnp.broadcast_to(t4[2], (16,)),
        jnp.broadcast_to(t4[3], (16,)),
    ])
    mesh = plsc.VectorSubcoreMesh(core_axis_name="c", subcore_axis_name="s")
    iou_flat = pl.kernel(
        _iou_body,
        mesh=mesh,
        out_type=jax.ShapeDtypeStruct((NPIX,), jnp.float32),
        scratch_types=[
            pltpu.VMEM((DIMS * CHUNK,), jnp.float32),
            pltpu.VMEM((2 * CHUNK,), jnp.float32),
            pltpu.VMEM((CHUNK,), jnp.float32),
            pltpu.VMEM((7 * 16,), jnp.float32),
            pltpu.SemaphoreType.DMA,
        ],
    )(feat, hwf, small)
    return iou_flat.reshape(W, W)


# speedup vs baseline: 3.7374x; 1.0286x over previous
import jax
import jax.numpy as jnp
from jax import lax
from jax.experimental import pallas as pl
from jax.experimental.pallas import tpu as pltpu
from jax.experimental.pallas import tpu_sc as plsc

DIMS = 4
W = 72
NPIX = W * W
NWORK = 27
CHUNK = NPIX // NWORK
NVEC = CHUNK // 16
RWIN = 10.0


def _iou_body(fused_hbm, hwf_hbm, out_hbm, feat_v, hw_v, out_v, small_v, sem):
    wid = lax.axis_index("s") * 2 + lax.axis_index("c")

    @pl.when(wid < NWORK)
    def _():
        base = wid * CHUNK
        cps = [
            pltpu.async_copy(fused_hbm.at[pl.ds(DIMS * NPIX, 7 * 16)],
                             small_v, sem),
            pltpu.async_copy(hwf_hbm.at[pl.ds(base, CHUNK)],
                             hw_v.at[pl.ds(0, CHUNK)], sem),
            pltpu.async_copy(hwf_hbm.at[pl.ds(NPIX + base, CHUNK)],
                             hw_v.at[pl.ds(CHUNK, CHUNK)], sem),
        ]
        for c in range(DIMS):
            cps.append(pltpu.async_copy(
                fused_hbm.at[pl.ds(c * NPIX + base, CHUNK)],
                feat_v.at[pl.ds(c * CHUNK, CHUNK)], sem))
        for cp in cps:
            cp.wait()
        chv = small_v[pl.ds(0, 16)]
        cwv = small_v[pl.ds(16, 16)]
        rad = small_v[pl.ds(32, 16)]
        tl = small_v[pl.ds(48, 16)]
        tr = small_v[pl.ds(64, 16)]
        tt = small_v[pl.ds(80, 16)]
        tb = small_v[pl.ds(96, 16)]
        for j in range(NVEC):
            hf = hw_v[pl.ds(j * 16, 16)]
            wf = hw_v[pl.ds(CHUNK + j * 16, 16)]
            rhf = hf - chv
            rwf = wf - cwv
            rha = jnp.abs(rhf)
            rwa = jnp.abs(rwf)
            inwin = (rha <= RWIN) & (rwa <= RWIN) & (rha <= rad) & (rwa <= rad)
            twl = tl + rwf
            twr = tr - rwf
            tht = tt + rhf
            thb = tb - rhf
            okt = (twl >= 0.0) & (twr >= 0.0) & (tht >= 0.0) & (thb >= 0.0)
            p0 = feat_v[pl.ds(0 * CHUNK + j * 16, 16)]
            p1 = feat_v[pl.ds(1 * CHUNK + j * 16, 16)]
            p2 = feat_v[pl.ds(2 * CHUNK + j * 16, 16)]
            p3 = feat_v[pl.ds(3 * CHUNK + j * 16, 16)]
            t_area = (twl + twr) * (tht + thb)
            p_area = (p0 + p1) * (p2 + p3)
            w_int = jnp.minimum(p0, twl) + jnp.minimum(p1, twr)
            h_int = jnp.minimum(p3, thb) + jnp.minimum(p2, tht)
            a_int = w_int * h_int
            a_un = t_area + p_area - a_int
            iou = (a_int + 1.0) / (a_un + 1.0)
            out_v[pl.ds(j * 16, 16)] = jnp.where(inwin & okt, iou, 0.0)
        pltpu.sync_copy(out_v, out_hbm.at[pl.ds(base, CHUNK)])


def kernel(output, ind, target, radius):
    feat = output.reshape(-1, DIMS * NPIX)[0]
    pix = jnp.arange(NPIX, dtype=jnp.float32)
    hwf = jnp.concatenate([jnp.floor(pix / W), pix % W])
    cen = ind.reshape(-1)[0].astype(jnp.float32)
    t4 = target.reshape(-1, DIMS)[0]
    fused = jnp.concatenate([
        feat,
        jnp.broadcast_to(jnp.floor(cen / W), (16,)),
        jnp.broadcast_to(cen % W, (16,)),
        jnp.broadcast_to(jnp.asarray(radius, jnp.float32), (16,)),
        jnp.broadcast_to(t4[0], (16,)),
        jnp.broadcast_to(t4[1], (16,)),
        jnp.broadcast_to(t4[2], (16,)),
        jnp.broadcast_to(t4[3], (16,)),
    ])
    mesh = plsc.VectorSubcoreMesh(core_axis_name="c", subcore_axis_name="s")
    iou_flat = pl.kernel(
        _iou_body,
        mesh=mesh,
        out_type=jax.ShapeDtypeStruct((NPIX,), jnp.float32),
        scratch_types=[
            pltpu.VMEM((DIMS * CHUNK,), jnp.float32),
            pltpu.VMEM((2 * CHUNK,), jnp.float32),
            pltpu.VMEM((CHUNK,), jnp.float32),
            pltpu.VMEM((7 * 16,), jnp.float32),
            pltpu.SemaphoreType.DMA,
        ],
    )(fused, hwf)
    return iou_flat.reshape(W, W)


# speedup vs baseline: 3.9335x; 1.0525x over previous
import jax
import jax.numpy as jnp
from jax import lax
from jax.experimental import pallas as pl
from jax.experimental.pallas import tpu as pltpu
from jax.experimental.pallas import tpu_sc as plsc

DIMS = 4
W = 72
NPIX = W * W
NWORK = 16
NPIXP = 5376
CHUNK = NPIXP // NWORK
NVEC = CHUNK // 16
RWIN = 10.0


def _iou_body(fused_hbm, hwf_hbm, out_hbm, feat_v, hw_v, out_v, small_v, sem):
    wid = lax.axis_index("s")
    base = wid * CHUNK
    cps = [
        pltpu.async_copy(fused_hbm.at[pl.ds(DIMS * NPIXP, 7 * 16)],
                         small_v, sem),
        pltpu.async_copy(hwf_hbm.at[pl.ds(base, CHUNK)],
                         hw_v.at[pl.ds(0, CHUNK)], sem),
        pltpu.async_copy(hwf_hbm.at[pl.ds(NPIXP + base, CHUNK)],
                         hw_v.at[pl.ds(CHUNK, CHUNK)], sem),
    ]
    for c in range(DIMS):
        cps.append(pltpu.async_copy(
            fused_hbm.at[pl.ds(c * NPIXP + base, CHUNK)],
            feat_v.at[pl.ds(c * CHUNK, CHUNK)], sem))
    for cp in cps:
        cp.wait()
    chv = small_v[pl.ds(0, 16)]
    cwv = small_v[pl.ds(16, 16)]
    rad = small_v[pl.ds(32, 16)]
    tl = small_v[pl.ds(48, 16)]
    tr = small_v[pl.ds(64, 16)]
    tt = small_v[pl.ds(80, 16)]
    tb = small_v[pl.ds(96, 16)]
    for j in range(NVEC):
        hf = hw_v[pl.ds(j * 16, 16)]
        wf = hw_v[pl.ds(CHUNK + j * 16, 16)]
        rhf = hf - chv
        rwf = wf - cwv
        rha = jnp.abs(rhf)
        rwa = jnp.abs(rwf)
        inwin = (rha <= RWIN) & (rwa <= RWIN) & (rha <= rad) & (rwa <= rad)
        twl = tl + rwf
        twr = tr - rwf
        tht = tt + rhf
        thb = tb - rhf
        okt = (twl >= 0.0) & (twr >= 0.0) & (tht >= 0.0) & (thb >= 0.0)
        p0 = feat_v[pl.ds(0 * CHUNK + j * 16, 16)]
        p1 = feat_v[pl.ds(1 * CHUNK + j * 16, 16)]
        p2 = feat_v[pl.ds(2 * CHUNK + j * 16, 16)]
        p3 = feat_v[pl.ds(3 * CHUNK + j * 16, 16)]
        t_area = (twl + twr) * (tht + thb)
        p_area = (p0 + p1) * (p2 + p3)
        w_int = jnp.minimum(p0, twl) + jnp.minimum(p1, twr)
        h_int = jnp.minimum(p3, thb) + jnp.minimum(p2, tht)
        a_int = w_int * h_int
        a_un = t_area + p_area - a_int
        iou = (a_int + 1.0) / (a_un + 1.0)
        out_v[pl.ds(j * 16, 16)] = jnp.where(inwin & okt, iou, 0.0)
    pltpu.sync_copy(out_v, out_hbm.at[pl.ds(base, CHUNK)])


def kernel(output, ind, target, radius):
    feat = output.reshape(-1, DIMS, NPIX)[0]
    featp = jnp.pad(feat, ((0, 0), (0, NPIXP - NPIX))).reshape(-1)
    pix = jnp.arange(NPIXP, dtype=jnp.float32)
    hwf = jnp.concatenate([jnp.floor(pix / W), pix % W])
    cen = ind.reshape(-1)[0].astype(jnp.float32)
    t4 = target.reshape(-1, DIMS)[0]
    fused = jnp.concatenate([
        featp,
        jnp.broadcast_to(jnp.floor(cen / W), (16,)),
        jnp.broadcast_to(cen % W, (16,)),
        jnp.broadcast_to(jnp.asarray(radius, jnp.float32), (16,)),
        jnp.broadcast_to(t4[0], (16,)),
        jnp.broadcast_to(t4[1], (16,)),
        jnp.broadcast_to(t4[2], (16,)),
        jnp.broadcast_to(t4[3], (16,)),
    ])
    mesh = plsc.VectorSubcoreMesh(
        core_axis_name="c", subcore_axis_name="s", num_cores=1)
    iou_flat = pl.kernel(
        _iou_body,
        mesh=mesh,
        out_type=jax.ShapeDtypeStruct((NPIXP,), jnp.float32),
        scratch_types=[
            pltpu.VMEM((DIMS * CHUNK,), jnp.float32),
            pltpu.VMEM((2 * CHUNK,), jnp.float32),
            pltpu.VMEM((CHUNK,), jnp.float32),
            pltpu.VMEM((7 * 16,), jnp.float32),
            pltpu.SemaphoreType.DMA,
        ],
    )(fused, hwf)
    return iou_flat[:NPIX].reshape(W, W)


# speedup vs baseline: 3.9508x; 1.0044x over previous
import jax
import jax.numpy as jnp
from jax import lax
from jax.experimental import pallas as pl
from jax.experimental.pallas import tpu as pltpu
from jax.experimental.pallas import tpu_sc as plsc

DIMS = 4
W = 72
NPIX = W * W
NWORK = 16
NPIXP = 5376
CHUNK = NPIXP // NWORK
NVEC = CHUNK // 16
NSMALL = 7 * 16
BLOCK = NSMALL + 6 * CHUNK
HW_OFF = NSMALL
FEAT_OFF = NSMALL + 2 * CHUNK
RWIN = 10.0


def _iou_body(fused_hbm, out_hbm, blk_v, out_v, sem):
    wid = lax.axis_index("s")
    pltpu.async_copy(fused_hbm.at[pl.ds(wid * BLOCK, BLOCK)], blk_v, sem).wait()
    chv = blk_v[pl.ds(0, 16)]
    cwv = blk_v[pl.ds(16, 16)]
    rad = blk_v[pl.ds(32, 16)]
    tl = blk_v[pl.ds(48, 16)]
    tr = blk_v[pl.ds(64, 16)]
    tt = blk_v[pl.ds(80, 16)]
    tb = blk_v[pl.ds(96, 16)]
    for j in range(NVEC):
        hf = blk_v[pl.ds(HW_OFF + j * 16, 16)]
        wf = blk_v[pl.ds(HW_OFF + CHUNK + j * 16, 16)]
        rhf = hf - chv
        rwf = wf - cwv
        rha = jnp.abs(rhf)
        rwa = jnp.abs(rwf)
        inwin = (rha <= RWIN) & (rwa <= RWIN) & (rha <= rad) & (rwa <= rad)
        twl = tl + rwf
        twr = tr - rwf
        tht = tt + rhf
        thb = tb - rhf
        okt = (twl >= 0.0) & (twr >= 0.0) & (tht >= 0.0) & (thb >= 0.0)
        p0 = blk_v[pl.ds(FEAT_OFF + 0 * CHUNK + j * 16, 16)]
        p1 = blk_v[pl.ds(FEAT_OFF + 1 * CHUNK + j * 16, 16)]
        p2 = blk_v[pl.ds(FEAT_OFF + 2 * CHUNK + j * 16, 16)]
        p3 = blk_v[pl.ds(FEAT_OFF + 3 * CHUNK + j * 16, 16)]
        t_area = (twl + twr) * (tht + thb)
        p_area = (p0 + p1) * (p2 + p3)
        w_int = jnp.minimum(p0, twl) + jnp.minimum(p1, twr)
        h_int = jnp.minimum(p3, thb) + jnp.minimum(p2, tht)
        a_int = w_int * h_int
        a_un = t_area + p_area - a_int
        iou = (a_int + 1.0) / (a_un + 1.0)
        out_v[pl.ds(j * 16, 16)] = jnp.where(inwin & okt, iou, 0.0)
    pltpu.sync_copy(out_v, out_hbm.at[pl.ds(wid * CHUNK, CHUNK)])


def kernel(output, ind, target, radius):
    feat = output.reshape(-1, DIMS, NPIX)[0]
    featp = jnp.pad(feat, ((0, 0), (0, NPIXP - NPIX)))
    feat_b = jnp.transpose(featp.reshape(DIMS, NWORK, CHUNK), (1, 0, 2))
    pix = jnp.arange(NPIXP, dtype=jnp.float32)
    hw = jnp.stack([jnp.floor(pix / W), pix % W])
    hw_b = jnp.transpose(hw.reshape(2, NWORK, CHUNK), (1, 0, 2))
    cen = ind.reshape(-1)[0].astype(jnp.float32)
    t4 = target.reshape(-1, DIMS)[0]
    small = jnp.concatenate([
        jnp.broadcast_to(jnp.floor(cen / W), (16,)),
        jnp.broadcast_to(cen % W, (16,)),
        jnp.broadcast_to(jnp.asarray(radius, jnp.float32), (16,)),
        jnp.broadcast_to(t4[0], (16,)),
        jnp.broadcast_to(t4[1], (16,)),
        jnp.broadcast_to(t4[2], (16,)),
        jnp.broadcast_to(t4[3], (16,)),
    ])
    fused = jnp.concatenate([
        jnp.broadcast_to(small[None, :], (NWORK, NSMALL)),
        hw_b.reshape(NWORK, 2 * CHUNK),
        feat_b.reshape(NWORK, DIMS * CHUNK),
    ], axis=1).reshape(-1)
    mesh = plsc.VectorSubcoreMesh(
        core_axis_name="c", subcore_axis_name="s", num_cores=1)
    iou_flat = pl.kernel(
        _iou_body,
        mesh=mesh,
        out_type=jax.ShapeDtypeStruct((NPIXP,), jnp.float32),
        scratch_types=[
            pltpu.VMEM((BLOCK,), jnp.float32),
            pltpu.VMEM((CHUNK,), jnp.float32),
            pltpu.SemaphoreType.DMA,
        ],
    )(fused)
    return iou_flat[:NPIX].reshape(W, W)
